# baseline (speedup 1.0000x reference)
import functools

import jax
import jax.numpy as jnp
import numpy as np
from jax import lax
from jax.experimental import pallas as pl
from jax.experimental.pallas import tpu as pltpu
from jax.experimental.pallas import tpu_sc as plsc

_FDIM = 128
_BATCH = 16384
_D = _FDIM // 2

_NC = 2
_NS = 16
_NW = _NC * _NS
_BPW = _BATCH // _NW
_IDXC = 128
_NCHUNK = _BPW // _IDXC
_G = 16
_GPC = _IDXC // _G

_N = 8192
_MAGIC = 12582912.0
_CD = [float((_N / (2.0 * np.pi)) / (10000.0 ** (d / (_D - 1)))) for d in range(_D)]


def _sc_fused(t_r, label_r, sintab, table):
    mesh = plsc.VectorSubcoreMesh(core_axis_name="c", subcore_axis_name="s")

    @functools.partial(
        pl.kernel,
        mesh=mesh,
        out_type=jax.ShapeDtypeStruct((_BATCH, _FDIM), jnp.float32),
        scratch_types=[
            pltpu.VMEM((_NCHUNK, _IDXC), jnp.int32),
            pltpu.VMEM((_BPW,), jnp.float32),
            pltpu.VMEM((_N,), jnp.float32),
            pltpu.VMEM((_BPW, _FDIM), jnp.float32),
            [pltpu.SemaphoreType.DMA] * _NCHUNK,
            pltpu.SemaphoreType.DMA,
        ],
        compiler_params=pltpu.CompilerParams(needs_layout_passes=False),
    )
    def k(t_hbm, label_hbm, sintab_hbm, table_hbm, out_hbm,
          idx_v, t_v, tab_v, rows_v, gsems, osem):
        wid = lax.axis_index("s") * _NC + lax.axis_index("c")
        base = wid * _BPW
        pltpu.sync_copy(label_hbm.at[wid], idx_v)
        gathers = [
            pltpu.async_copy(
                table_hbm.at[idx_v.at[j]],
                rows_v.at[pl.ds(j * _IDXC, _IDXC)],
                gsems[j],
            )
            for j in range(_NCHUNK)
        ]
        pltpu.sync_copy(t_hbm.at[wid], t_v)
        pltpu.sync_copy(sintab_hbm, tab_v)
        lane = lax.iota(jnp.int32, _G)
        out_copies = []
        for j in range(_NCHUNK):
            gathers[j].wait()

            def body(g, _, j=j):
                e0 = j * _IDXC + g * _G
                tvec = t_v[pl.ds(e0, _G)]
                row_idx = lane + e0
                for d in range(_D):
                    kf0 = tvec * _CD[d] + _MAGIC
                    ki = kf0.astype(jnp.int32)
                    idx_s = ki & (_N - 1)
                    idx_c = (ki + _N // 4) & (_N - 1)
                    sin_v = plsc.load_gather(tab_v, [idx_s])
                    cos_v = plsc.load_gather(tab_v, [idx_c])
                    col_s = jnp.full((_G,), d, jnp.int32)
                    col_c = jnp.full((_G,), d + _D, jnp.int32)
                    plsc.addupdate_scatter(rows_v, [row_idx, col_s], sin_v)
                    plsc.addupdate_scatter(rows_v, [row_idx, col_c], cos_v)
                return 0

            lax.fori_loop(0, _GPC, body, 0)
            out_copies.append(
                pltpu.async_copy(
                    rows_v.at[pl.ds(j * _IDXC, _IDXC)],
                    out_hbm.at[pl.ds(base + j * _IDXC, _IDXC)],
                    osem,
                )
            )
        for c in out_copies:
            c.wait()

    return k(t_r, label_r, sintab, table)


def kernel(t, label, class_emb):
    label_r = label.astype(jnp.int32).reshape(_NW, _NCHUNK, _IDXC)
    t_r = t.reshape(_NW, _BPW)
    sintab = jnp.sin(
        jnp.arange(_N, dtype=jnp.float32) * jnp.float32(2.0 * np.pi / _N)
    )
    return _sc_fused(t_r, label_r, sintab, class_emb)


# speedup vs baseline: 1.6025x; 1.6025x over previous
import functools

import jax
import jax.numpy as jnp
import numpy as np
from jax import lax
from jax.experimental import pallas as pl
from jax.experimental.pallas import tpu as pltpu
from jax.experimental.pallas import tpu_sc as plsc

_FDIM = 128
_BATCH = 16384
_D = _FDIM // 2

_NC = 2
_NS = 16
_NW = _NC * _NS
_BPW = _BATCH // _NW
_IDXC = 128
_NCHUNK = _BPW // _IDXC
_G = 16
_GPC = _IDXC // _G
_DC = _D // _G

_N = 8192
_MAGIC = 12582912.0
_CD = [float((_N / (2.0 * np.pi)) / (10000.0 ** (d / (_D - 1)))) for d in range(_D)]


def _sc_fused(t_r, label_r, cd, sintab, table):
    mesh = plsc.VectorSubcoreMesh(core_axis_name="c", subcore_axis_name="s")

    @functools.partial(
        pl.kernel,
        mesh=mesh,
        out_type=jax.ShapeDtypeStruct((_BATCH, _FDIM), jnp.float32),
        scratch_types=[
            pltpu.VMEM((_NCHUNK, _IDXC), jnp.int32),
            pltpu.VMEM((_BPW,), jnp.float32),
            pltpu.VMEM((_DC, _G), jnp.float32),
            pltpu.VMEM((_N,), jnp.float32),
            pltpu.VMEM((_BPW, _FDIM), jnp.float32),
            [pltpu.SemaphoreType.DMA] * _NCHUNK,
            pltpu.SemaphoreType.DMA,
        ],
        compiler_params=pltpu.CompilerParams(needs_layout_passes=False),
    )
    def k(t_hbm, label_hbm, cd_hbm, sintab_hbm, table_hbm, out_hbm,
          idx_v, t_v, cd_v, tab_v, rows_v, gsems, osem):
        wid = lax.axis_index("s") * _NC + lax.axis_index("c")
        base = wid * _BPW
        pltpu.sync_copy(label_hbm.at[wid], idx_v)
        gathers = [
            pltpu.async_copy(
                table_hbm.at[idx_v.at[j]],
                rows_v.at[pl.ds(j * _IDXC, _IDXC)],
                gsems[j],
            )
            for j in range(_NCHUNK)
        ]
        pltpu.sync_copy(t_hbm.at[wid], t_v)
        pltpu.sync_copy(cd_hbm, cd_v)
        pltpu.sync_copy(sintab_hbm, tab_v)
        cds = [cd_v[c] for c in range(_DC)]
        out_copies = []
        for j in range(_NCHUNK):
            gathers[j].wait()

            def body(g, _, j=j):
                e0 = j * _IDXC + g * _G
                tvec = t_v[pl.ds(e0, _G)]
                for i in range(_G):
                    tv = tvec[i]
                    e = e0 + i
                    for c in range(_DC):
                        kf0 = tv * cds[c] + _MAGIC
                        ki = kf0.astype(jnp.int32)
                        idx_s = ki & (_N - 1)
                        idx_c = (ki + _N // 4) & (_N - 1)
                        sin_v = plsc.load_gather(tab_v, [idx_s])
                        cos_v = plsc.load_gather(tab_v, [idx_c])
                        lo = pl.ds(c * _G, _G)
                        hi = pl.ds(_D + c * _G, _G)
                        rows_v[e, lo] = rows_v[e, lo] + sin_v
                        rows_v[e, hi] = rows_v[e, hi] + cos_v
                return 0

            lax.fori_loop(0, _GPC, body, 0)
            out_copies.append(
                pltpu.async_copy(
                    rows_v.at[pl.ds(j * _IDXC, _IDXC)],
                    out_hbm.at[pl.ds(base + j * _IDXC, _IDXC)],
                    osem,
                )
            )
        for c in out_copies:
            c.wait()

    return k(t_r, label_r, cd, sintab, table)


def kernel(t, label, class_emb):
    label_r = label.astype(jnp.int32).reshape(_NW, _NCHUNK, _IDXC)
    t_r = t.reshape(_NW, _BPW)
    cd = jnp.asarray(_CD, dtype=jnp.float32).reshape(_DC, _G)
    sintab = jnp.sin(
        jnp.arange(_N, dtype=jnp.float32) * jnp.float32(2.0 * np.pi / _N)
    )
    return _sc_fused(t_r, label_r, cd, sintab, class_emb)


# speedup vs baseline: 1.8141x; 1.1320x over previous
import functools

import jax
import jax.numpy as jnp
from jax import lax
from jax.experimental import pallas as pl
from jax.experimental.pallas import tpu as pltpu
from jax.experimental.pallas import tpu_sc as plsc

_FDIM = 128
_BATCH = 16384
_D = _FDIM // 2

_NC = 2
_NS = 16
_NW = _NC * _NS
_BPW = _BATCH // _NW
_IDXC = 128
_NCHUNK = _BPW // _IDXC

_MAGIC = 12582912.0
_INV2PI = 0.15915494309189535
_HI = 6.28125
_LO = 0.0019353071795864769
_S = (0.9994499856355539, -0.1658382205956817, 0.007998520295566539,
      -0.00014773645596885543)
_C = (0.9999710807349519, -0.49983754043485695, 0.04152226790055865,
      -0.001344099441249913, 1.9064759252396257e-05)


def _sc_gather(label_r, table):
    mesh = plsc.VectorSubcoreMesh(core_axis_name="c", subcore_axis_name="s")

    @functools.partial(
        pl.kernel,
        mesh=mesh,
        out_type=jax.ShapeDtypeStruct((_BATCH, _FDIM), jnp.float32),
        scratch_types=[
            pltpu.VMEM((_NCHUNK, _IDXC), jnp.int32),
            pltpu.VMEM((_BPW, _FDIM), jnp.float32),
            [pltpu.SemaphoreType.DMA] * _NCHUNK,
            pltpu.SemaphoreType.DMA,
        ],
    )
    def k(label_hbm, table_hbm, out_hbm, idx_v, rows_v, gsems, osem):
        wid = lax.axis_index("s") * _NC + lax.axis_index("c")
        base = wid * _BPW
        pltpu.sync_copy(label_hbm.at[wid], idx_v)
        gathers = [
            pltpu.async_copy(
                table_hbm.at[idx_v.at[j]],
                rows_v.at[pl.ds(j * _IDXC, _IDXC)],
                gsems[j],
            )
            for j in range(_NCHUNK)
        ]
        out_copies = []
        for j in range(_NCHUNK):
            gathers[j].wait()
            out_copies.append(
                pltpu.async_copy(
                    rows_v.at[pl.ds(j * _IDXC, _IDXC)],
                    out_hbm.at[pl.ds(base + j * _IDXC, _IDXC)],
                    osem,
                )
            )
        for c in out_copies:
            c.wait()

    return k(label_r, table)


def _tc_body(t_ref, invd_ref, g_ref, o_ref):
    x = t_ref[...] * invd_ref[...]
    kf = jnp.floor(x * _INV2PI + 0.5)
    r = (x - kf * _HI) - kf * _LO
    z = r * r
    sp = _S[-1]
    for a in _S[-2::-1]:
        sp = sp * z + a
    sin_v = sp * r
    cp = _C[-1]
    for a in _C[-2::-1]:
        cp = cp * z + a
    emb = jnp.concatenate((sin_v, cp), axis=1)
    o_ref[...] = emb + g_ref[...]


def _tc_sin_add(t2, invd2, g):
    bb = 4096
    return pl.pallas_call(
        _tc_body,
        out_shape=jax.ShapeDtypeStruct((_BATCH, _FDIM), jnp.float32),
        grid=(_BATCH // bb,),
        in_specs=[
            pl.BlockSpec((bb, 1), lambda i: (i, 0)),
            pl.BlockSpec((1, _D), lambda i: (0, 0)),
            pl.BlockSpec((bb, _FDIM), lambda i: (i, 0)),
        ],
        out_specs=pl.BlockSpec((bb, _FDIM), lambda i: (i, 0)),
    )(t2, invd2, g)


def kernel(t, label, class_emb):
    label_r = label.astype(jnp.int32).reshape(_NW, _NCHUNK, _IDXC)
    gathered = _sc_gather(label_r, class_emb)
    denom = 10000.0 ** (jnp.arange(_D, dtype=jnp.float32) / (_D - 1))
    invd = (1.0 / denom).reshape(1, _D)
    return _tc_sin_add(t.reshape(_BATCH, 1), invd, gathered)
